# baseline (speedup 1.0000x reference)
import functools

import jax
import jax.numpy as jnp
from jax import lax
from jax.experimental import pallas as pl
from jax.experimental.pallas import tpu as pltpu
from jax.experimental.pallas import tpu_sc as plsc

NUM_CLASSES = 1000000
FEAT_DIM = 32
BATCH = 16384

NC = 2
NS = 16
NW = NC * NS
B_PER_W = BATCH // NW
IDX_CHUNK = 128
N_CHUNKS = B_PER_W // IDX_CHUNK

_mesh = plsc.VectorSubcoreMesh(core_axis_name="c", subcore_axis_name="s")


@functools.partial(
    pl.kernel,
    mesh=_mesh,
    compiler_params=pltpu.CompilerParams(use_tc_tiling_on_sc=False),
    out_type=jax.ShapeDtypeStruct((NW, 16), jnp.float32),
    scratch_types=[
        pltpu.VMEM((N_CHUNKS, IDX_CHUNK), jnp.int32),
        pltpu.VMEM((B_PER_W, FEAT_DIM), jnp.float32),
        pltpu.VMEM((B_PER_W, FEAT_DIM), jnp.float32),
        pltpu.VMEM((16,), jnp.float32),
        pltpu.SemaphoreType.DMA,
    ],
)
def _center_loss_kernel(x_hbm, labels_hbm, centers_hbm, out_hbm,
                        idx_v, rows_v, x_v, acc_v, sem):
    wid = lax.axis_index("s") * NC + lax.axis_index("c")

    pltpu.sync_copy(labels_hbm.at[wid], idx_v)

    copies = [
        pltpu.async_copy(
            centers_hbm.at[idx_v.at[j]],
            rows_v.at[pl.ds(j * IDX_CHUNK, IDX_CHUNK)],
            sem,
        )
        for j in range(N_CHUNKS)
    ]
    pltpu.sync_copy(x_hbm.at[wid], x_v)
    for c in copies:
        c.wait()

    def body(i, acc):
        for h in (0, 16):
            d = x_v[i, pl.ds(h, 16)] - rows_v[i, pl.ds(h, 16)]
            acc = acc + d * d
        return acc

    acc = lax.fori_loop(0, B_PER_W, body, jnp.zeros((16,), jnp.float32))
    acc_v[...] = acc
    pltpu.sync_copy(acc_v, out_hbm.at[wid])


def kernel(x, labels, centers):
    labels3 = labels.astype(jnp.int32).reshape(NW, N_CHUNKS, IDX_CHUNK)
    x3 = x.reshape(NW, B_PER_W, FEAT_DIM)
    partials = _center_loss_kernel(x3, labels3, centers)
    return jnp.sum(partials)
